# baseline (speedup 1.0000x reference)
import functools

import jax
import jax.numpy as jnp
from jax import lax
from jax.experimental import pallas as pl
from jax.experimental.pallas import tpu as pltpu
from jax.experimental.pallas import tpu_sc as plsc

D = 64
B = 16384
NC = 2
NS = 16
NW = NC * NS
L = 16
UW = 512
SLOTS = 64
ROW_BYTES = D * 4

N_USER = 1000000
N_ITEM = 100096
N_TAG = 1024


@functools.partial(
    pl.kernel,
    mesh=plsc.VectorSubcoreMesh(core_axis_name="c", subcore_axis_name="s"),
    out_type=(
        jax.ShapeDtypeStruct((B, D), jnp.float32),
        jax.ShapeDtypeStruct((B, D), jnp.float32),
        jax.ShapeDtypeStruct((B, D), jnp.float32),
        jax.ShapeDtypeStruct((L, D), jnp.float32),
    ),
    scratch_types=[
        pltpu.VMEM((D, UW), jnp.float32),
        pltpu.VMEM((D, UW), jnp.float32),
        pltpu.VMEM((D, 256), jnp.float32),
        pltpu.VMEM((B + L,), jnp.int32),
        pltpu.VMEM((B,), jnp.int32),
        pltpu.VMEM((SLOTS, D), jnp.float32),
        pltpu.SemaphoreType.DMA,
        pltpu.SemaphoreType.DMA,
        pltpu.SemaphoreType.DMA,
    ],
    compiler_params=pltpu.CompilerParams(needs_layout_passes=False),
)
def _gather3(eu, eutail, ei, et, iu, ii, it, ou, oi, ot, otr,
             buf0, buf1, tbuf, wl, idxb, rowbuf, s0, s1, osem):
    wid = lax.axis_index("s") * NC + lax.axis_index("c")
    iota = lax.iota(jnp.int32, L)
    dvec = [iota + 16 * j for j in range(4)]
    bufs = (buf0, buf1)
    sems = (s0, s1)

    def scan(idx, k0, klen, c0, cend):
        pltpu.sync_copy(idx.at[pl.ds(k0, klen)], idxb.at[pl.ds(0, klen)])

        def body(m, cnt):
            v = idxb[pl.ds(m * L, L)]
            tc = lax.shift_right_logical(v, 7)
            mask = (tc >= c0) & (tc < cend)
            p = ((tc - c0) << 21) | ((v & 127) << 14) | (k0 + m * L + iota)
            mski = jnp.where(mask, 1, 0)
            pos = plsc.cumsum(mski) - mski
            plsc.store_scatter(wl, [cnt + pos], p, mask=mask)
            n = plsc.all_reduce_population_count(mask)[0]
            return cnt + n

        return lax.fori_loop(0, klen // L, body, jnp.int32(0))

    def process_unit(buf, su_rel, cnt, out, iss_wtd):
        def chunk(m, carry):
            iss, wtd = carry
            valid = (m * L + iota) < cnt
            p = wl[pl.ds(m * L, L)]
            brel = lax.shift_right_logical(p, 21)
            match = (lax.shift_right_logical(brel, 2) == su_rel) & valid
            npop = plsc.all_reduce_population_count(match)[0]
            mi = jnp.where(match, 1, 0)
            pos = plsc.cumsum(mi) - mi
            do_wait = (iss - wtd + npop) > SLOTS

            @pl.when(do_wait)
            def _():
                pltpu.make_async_copy(
                    ou.at[pl.ds(0, L), :], rowbuf.at[pl.ds(0, L), :], osem
                ).wait()

            wtd = jnp.where(do_wait, wtd + L, wtd)

            @pl.when(npop > 0)
            def _():
                col = ((brel & 3) << 7) | (lax.shift_right_logical(p, 14) & 127)
                kk = p & 0x3FFF
                for l in range(L):
                    @pl.when(mi[l] == 1)
                    def _(l=l):
                        slot = (iss + pos[l]) % SLOTS
                        cs = jnp.full((L,), col[l], jnp.int32)
                        for j in range(4):
                            rowbuf[slot, pl.ds(16 * j, L)] = plsc.load_gather(
                                buf, [dvec[j], cs])
                        pltpu.make_async_copy(
                            rowbuf.at[pl.ds(slot, 1), :],
                            out.at[pl.ds(kk[l], 1), :],
                            osem,
                        ).start()

            return (iss + npop, wtd)

        nchunks = (cnt + L - 1) // L
        return lax.fori_loop(0, nchunks, chunk, iss_wtd)

    def drain(iss_wtd):
        iss, wtd = iss_wtd
        pad = (-iss) % L
        for i in range(L - 1):
            @pl.when(pad > i)
            def _():
                pltpu.make_async_copy(
                    rowbuf.at[pl.ds(0, 1), :], otr.at[pl.ds(0, 1), :], osem
                ).start()

        iss = iss + pad
        for i in range(SLOTS // L):
            @pl.when(iss - wtd > i * L)
            def _():
                pltpu.make_async_copy(
                    ou.at[pl.ds(0, L), :], rowbuf.at[pl.ds(0, L), :], osem
                ).wait()

    def do_regions(tab, idx, out, nu_full):
        u0 = wid * (nu_full) // NW
        u1 = (wid + 1) * (nu_full) // NW
        cnt = scan(idx, 0, B, u0 * 4, u1 * 4)

        @pl.when(u0 < u1)
        def _():
            pltpu.make_async_copy(
                tab.at[:, pl.ds(u0 * UW, UW)], buf0, s0).start()

        def pair(t2, carry):
            out_carry = carry
            for par in range(2):
                b = t2 * 2 + par
                su = u0 + b

                @pl.when(su + 1 < u1)
                def _(par=par, su=su):
                    pltpu.make_async_copy(
                        tab.at[:, pl.ds((su + 1) * UW, UW)],
                        bufs[1 - par], sems[1 - par]).start()

                cond = su < u1
                iss, wtd = out_carry

                @pl.when(cond)
                def _(par=par):
                    pltpu.make_async_copy(
                        tab.at[:, pl.ds(0, UW)], bufs[par], sems[par]).wait()

                iss2, wtd2 = process_unit(bufs[par],
                                          jnp.where(cond, su - u0, -1),
                                          cnt, out, (iss, wtd))
                out_carry = (iss2, wtd2)
            return out_carry

        npairs = (u1 - u0 + 1) // 2
        iss_wtd = lax.fori_loop(0, npairs, pair, (jnp.int32(0), jnp.int32(0)))
        drain(iss_wtd)
        return u1

    u1 = do_regions(eu, iu, ou, 1953)

    @pl.when(u1 == 1953)
    def _():
        cnt = scan(iu, 0, B, 7812, 7813)
        pltpu.make_async_copy(eutail, tbuf.at[:, pl.ds(0, 128)], s0).start()
        pltpu.make_async_copy(eutail, tbuf.at[:, pl.ds(0, 128)], s0).wait()
        iss_wtd = process_unit(tbuf, 0, cnt, ou, (jnp.int32(0), jnp.int32(0)))
        drain(iss_wtd)

    u1 = do_regions(ei, ii, oi, 195)

    @pl.when(u1 == 195)
    def _():
        cnt = scan(ii, 0, B, 780, 782)
        pltpu.make_async_copy(ei.at[:, pl.ds(780 * 128, 256)],
                              tbuf, s0).start()
        pltpu.make_async_copy(ei.at[:, pl.ds(780 * 128, 256)],
                              tbuf, s0).wait()
        iss_wtd = process_unit(tbuf, 0, cnt, oi, (jnp.int32(0), jnp.int32(0)))
        drain(iss_wtd)

    tu = wid & 1
    tk0 = lax.shift_right_logical(wid, 1) * (B // 16)
    cnt = scan(it, tk0, B // 16, tu * 4, tu * 4 + 4)
    pltpu.make_async_copy(et.at[:, pl.ds(tu * UW, UW)], buf0, s0).start()
    pltpu.make_async_copy(et.at[:, pl.ds(tu * UW, UW)], buf0, s0).wait()
    iss_wtd = process_unit(buf0, 0, cnt, ot, (jnp.int32(0), jnp.int32(0)))
    drain(iss_wtd)


def kernel(embed_user, embed_item, embed_tag, idx_user, idx_item, idx_tag):
    eut = embed_user.T
    eutail = jnp.pad(eut[:, 7812 * 128:], ((0, 0), (0, 64)))
    eit = jnp.pad(embed_item.T, ((0, 0), (0, 96)))
    ett = jnp.pad(embed_tag.T, ((0, 0), (0, 24)))
    ou, oi, ot, _ = _gather3(eut, eutail, eit, ett,
                             idx_user, idx_item, idx_tag)
    return (ou, oi, ot)


# speedup vs baseline: 1.1763x; 1.1763x over previous
import functools

import jax
import jax.numpy as jnp
from jax import lax
from jax.experimental import pallas as pl
from jax.experimental.pallas import tpu as pltpu
from jax.experimental.pallas import tpu_sc as plsc

D = 64
B = 16384
NC = 2
NS = 16
NW = NC * NS
L = 16
UW = 256
SLOTS = 32
ROW_BYTES = D * 4

N_USER = 1000000
N_ITEM = 100096
N_TAG = 1024


@functools.partial(
    pl.kernel,
    mesh=plsc.VectorSubcoreMesh(core_axis_name="c", subcore_axis_name="s"),
    out_type=(
        jax.ShapeDtypeStruct((B, D), jnp.float32),
        jax.ShapeDtypeStruct((B, D), jnp.float32),
        jax.ShapeDtypeStruct((B, D), jnp.float32),
        jax.ShapeDtypeStruct((L, D), jnp.float32),
    ),
    scratch_types=[
        pltpu.VMEM((D, UW), jnp.float32),
        pltpu.VMEM((D, UW), jnp.float32),
        pltpu.VMEM((B + L,), jnp.int32),
        pltpu.VMEM((2048,), jnp.int32),
        pltpu.VMEM((B // NW, D), jnp.float32),
        pltpu.VMEM((SLOTS, D), jnp.float32),
        pltpu.SemaphoreType.DMA,
        pltpu.SemaphoreType.DMA,
        pltpu.SemaphoreType.DMA,
    ],
    compiler_params=pltpu.CompilerParams(needs_layout_passes=False),
)
def _gather3(eu, eutail, ei, et, iu, ii, it, ou, oi, ot, otr,
             buf0, buf1, wl, idxb, rows, rowbuf, s0, s1, osem):
    wid = lax.axis_index("s") * NC + lax.axis_index("c")
    iota = lax.iota(jnp.int32, L)
    dvec = [iota + 16 * j for j in range(4)]
    bufs = (buf0, buf1)
    sems = (s0, s1)

    def scan(idx, k0, klen, c0, cend):
        def outer(g, gcnt):
            pltpu.sync_copy(idx.at[pl.ds(k0 + g * 2048, 2048)], idxb)

            def body(m, cnt):
                v = idxb[pl.ds(m * L, L)]
                tc = lax.shift_right_logical(v, 7)
                mask = (tc >= c0) & (tc < cend)
                p = (((tc - c0) << 21) | ((v & 127) << 14)
                     | (k0 + g * 2048 + m * L + iota))
                mski = jnp.where(mask, 1, 0)
                incl = plsc.cumsum(mski)
                plsc.store_scatter(wl, [cnt + incl - mski], p, mask=mask)
                return cnt + incl[L - 1]

            return lax.fori_loop(0, 2048 // L, body, gcnt)

        return lax.fori_loop(0, klen // 2048, outer, jnp.int32(0))

    def process_unit(buf, su_rel, cnt, out, iss_wtd):
        def chunk(m, carry):
            iss, wtd = carry
            valid = (m * L + iota) < cnt
            p = wl[pl.ds(m * L, L)]
            brel = lax.shift_right_logical(p, 21)
            match = (lax.shift_right_logical(brel, 1) == su_rel) & valid
            mi = jnp.where(match, 1, 0)
            incl = plsc.cumsum(mi)
            pos = incl - mi
            npop = incl[L - 1]
            do_wait = (iss - wtd + npop) > SLOTS

            @pl.when(do_wait)
            def _():
                pltpu.make_async_copy(
                    ou.at[pl.ds(0, L), :], rowbuf.at[pl.ds(0, L), :], osem
                ).wait()

            wtd = jnp.where(do_wait, wtd + L, wtd)

            @pl.when(npop > 0)
            def _():
                col = ((brel & 1) << 7) | (lax.shift_right_logical(p, 14) & 127)
                kk = p & 0x3FFF
                for l in range(L):
                    @pl.when(mi[l] == 1)
                    def _(l=l):
                        slot = (iss + pos[l]) % SLOTS
                        cs = jnp.full((L,), col[l], jnp.int32)
                        for j in range(4):
                            rowbuf[slot, pl.ds(16 * j, L)] = plsc.load_gather(
                                buf, [dvec[j], cs])
                        pltpu.make_async_copy(
                            rowbuf.at[pl.ds(slot, 1), :],
                            out.at[pl.ds(kk[l], 1), :],
                            osem,
                        ).start()

            return (iss + npop, wtd)

        nchunks = (cnt + L - 1) // L
        return lax.fori_loop(0, nchunks, chunk, iss_wtd)

    def drain(iss_wtd):
        iss, wtd = iss_wtd
        pad = (-iss) % L
        for i in range(L - 1):
            @pl.when(pad > i)
            def _():
                pltpu.make_async_copy(
                    rowbuf.at[pl.ds(0, 1), :], otr.at[pl.ds(0, 1), :], osem
                ).start()

        iss = iss + pad
        for i in range(SLOTS // L):
            @pl.when(iss - wtd > i * L)
            def _():
                pltpu.make_async_copy(
                    ou.at[pl.ds(0, L), :], rowbuf.at[pl.ds(0, L), :], osem
                ).wait()

    def do_regions(tab, idx, out, nu_full):
        u0 = wid * (nu_full) // NW
        u1 = (wid + 1) * (nu_full) // NW
        cnt = scan(idx, 0, B, u0 * 2, u1 * 2)

        @pl.when(u0 < u1)
        def _():
            pltpu.make_async_copy(
                tab.at[:, pl.ds(u0 * UW, UW)], buf0, s0).start()

        def pair(t2, carry):
            out_carry = carry
            for par in range(2):
                b = t2 * 2 + par
                su = u0 + b

                @pl.when(su + 1 < u1)
                def _(par=par, su=su):
                    pltpu.make_async_copy(
                        tab.at[:, pl.ds((su + 1) * UW, UW)],
                        bufs[1 - par], sems[1 - par]).start()

                cond = su < u1
                iss, wtd = out_carry

                @pl.when(cond)
                def _(par=par):
                    pltpu.make_async_copy(
                        tab.at[:, pl.ds(0, UW)], bufs[par], sems[par]).wait()

                iss2, wtd2 = process_unit(bufs[par],
                                          jnp.where(cond, su - u0, -1),
                                          cnt, out, (iss, wtd))
                out_carry = (iss2, wtd2)
            return out_carry

        npairs = (u1 - u0 + 1) // 2
        iss_wtd = lax.fori_loop(0, npairs, pair, (jnp.int32(0), jnp.int32(0)))
        drain(iss_wtd)
        return u1

    u1 = do_regions(eu, iu, ou, 3906)

    @pl.when(u1 == 3906)
    def _():
        cnt = scan(iu, 0, B, 7812, 7813)
        pltpu.make_async_copy(eutail, buf0.at[:, pl.ds(0, 128)], s0).start()
        pltpu.make_async_copy(eutail, buf0.at[:, pl.ds(0, 128)], s0).wait()
        iss_wtd = process_unit(buf0, 0, cnt, ou, (jnp.int32(0), jnp.int32(0)))
        drain(iss_wtd)

    base = wid * (B // NW)
    sl = pl.ds(base, B // NW)
    for tab, idx, out in ((ei, ii, oi), (et, it, ot)):
        pltpu.sync_copy(idx.at[sl], idxb.at[pl.ds(0, B // NW)])

        def rbody(m, _):
            vals = idxb[pl.ds(m * L, L)]
            for l in range(L):
                i = vals[l]
                pltpu.make_async_copy(
                    tab.at[pl.ds(i, 1), :],
                    rows.at[pl.ds(m * L + l, 1), :],
                    s1,
                ).start()
            return 0

        lax.fori_loop(0, B // NW // L, rbody, 0)
        pltpu.make_async_copy(tab.at[pl.ds(0, B // NW), :], rows, s1).wait()
        pltpu.async_copy(rows, out.at[sl], osem).wait()


def kernel(embed_user, embed_item, embed_tag, idx_user, idx_item, idx_tag):
    eut = embed_user.T
    eutail = jnp.pad(eut[:, 7812 * 128:], ((0, 0), (0, 64)))
    ou, oi, ot, _ = _gather3(eut, eutail, embed_item, embed_tag,
                             idx_user, idx_item, idx_tag)
    return (ou, oi, ot)
